# baseline (speedup 1.0000x reference)
import numpy as np

import jax
import jax.numpy as jnp
from jax.experimental import pallas as pl
from jax.experimental.pallas import tpu as pltpu


_B = np.zeros((32, 2, 14, 5), np.float32)
for _br in range(2):
    for _col2 in range(14):
        for _dj in range(5):
            _ci = 2 * _col2 + _br + _dj
            if _ci < 32:
                _B[_ci, _br, _col2, _dj] = 1.0


def _build_conv1_weights(c1_w):
    w4 = c1_w[:, :15, :6].reshape(5, 5, 3, 6)
    t = jnp.einsum("DJco,CbKJ->DcCbKo", w4, _B)
    t = jnp.pad(t, ((0, 0),) * 5 + ((0, 2),))
    t = t.reshape(5, 96, 2, 112)
    t = jnp.pad(t, ((0, 0), (0, 0), (0, 0), (0, 16)))
    return t.reshape(5, 96, 256).astype(jnp.bfloat16)


def _build_conv2_weights(c2_w):
    z = jnp.zeros((128, 256), c2_w.dtype)
    w = jnp.stack([
        jnp.concatenate([c2_w[0], c2_w[1]], axis=0),
        jnp.concatenate([c2_w[2], c2_w[3]], axis=0),
        jnp.concatenate([c2_w[4], z], axis=0),
        jnp.concatenate([z, c2_w[0]], axis=0),
        jnp.concatenate([c2_w[1], c2_w[2]], axis=0),
        jnp.concatenate([c2_w[3], c2_w[4]], axis=0),
    ])
    return w.astype(jnp.bfloat16)


def _build_fc1_weights(fc1_w):
    def ch(r):
        return jnp.pad(fc1_w[80 * r:80 * r + 80], ((0, 48), (0, 0)))
    z = jnp.zeros((128, 128), fc1_w.dtype)
    w = jnp.stack([
        jnp.concatenate([ch(0), ch(1)], axis=0),
        jnp.concatenate([ch(2), ch(3)], axis=0),
        jnp.concatenate([ch(4), z], axis=0),
    ])
    return w.astype(jnp.bfloat16)


def _fused_kernel(x_ref, w1_ref, b1_ref, w2_ref, b2_ref, wf_ref, bf1_ref,
                  w2f_ref, bf2_ref, w3f_ref, bf3_ref, o_ref,
                  xsc, gsc, h1sc):
    bt = x_ref.shape[0]
    R4 = bt * 32
    R = bt * 8
    f32 = jnp.float32
    bf16 = jnp.bfloat16

    xsc[pl.ds(R4, 8), :] = jnp.zeros((8, 96), bf16)
    for c in range(3):
        xc = x_ref[:, c].reshape(R4, 32).astype(bf16)
        xsc[pl.ds(0, R4), pl.ds(c * 32, 32)] = xc

    ch = min(512, R4)
    for s in range(R4 // ch):
        base = s * ch
        acc = None
        for di in range(5):
            lhs = xsc[pl.ds(base + di, ch), :]
            d = jnp.dot(lhs, w1_ref[di], preferred_element_type=f32)
            acc = d if acc is None else acc + d
        m = jnp.maximum(acc[:, :128], acc[:, 128:])
        gh = jnp.maximum(m + b1_ref[...], 0.0).astype(bf16)
        gsc[pl.ds(base // 32, ch // 32), :, :] = gh.reshape(ch // 32, 32, 128)

    for u in range(8):
        ev = jnp.maximum(gsc[:, 4 * u, :], gsc[:, 4 * u + 1, :])
        od = jnp.maximum(gsc[:, 4 * u + 2, :], gsc[:, 4 * u + 3, :])
        h1sc[:, u, pl.ds(0, 128)] = ev
        h1sc[:, u, pl.ds(128, 128)] = od

    h1 = h1sc[...].reshape(R, 256)
    accs = [None, None]
    for o in range(3):
        sl = h1[o:R - 2 + o]
        for p in range(2):
            d = jnp.dot(sl, w2_ref[p * 3 + o], preferred_element_type=f32)
            accs[p] = d if accs[p] is None else accs[p] + d
    m2 = jnp.maximum(accs[0], accs[1])
    m2 = jnp.maximum(m2[:, :128], m2[:, 128:])
    h2 = jnp.maximum(m2 + b2_ref[...], 0.0).astype(bf16)
    h2 = jnp.concatenate([h2, jnp.zeros((2, 128), bf16)], axis=0)
    h2 = h2.reshape(bt, 8, 128)

    f = None
    for r in range(3):
        l = jnp.concatenate([h2[:, 2 * r, :], h2[:, 2 * r + 1, :]], axis=1)
        d = jnp.dot(l, wf_ref[r], preferred_element_type=f32)
        f = d if f is None else f + d
    h = jnp.maximum(f + bf1_ref[...], 0.0).astype(bf16)
    h = jnp.dot(h, w2f_ref[...], preferred_element_type=f32)
    h = jnp.maximum(h + bf2_ref[...], 0.0).astype(bf16)
    out = jnp.dot(h, w3f_ref[...], preferred_element_type=f32) + bf3_ref[...]
    o_ref[...] = out


def kernel(c1_w, c1_b, c2_w, c2_b, fc1_w, fc1_b, fc2_w, fc2_b, fc3_w, fc3_b,
           x):
    B = x.shape[0]
    bt = next(b for b in (128, 64, 32, 16, 8, 4, 2, 1) if B % b == 0)

    w1 = _build_conv1_weights(c1_w)
    w2 = _build_conv2_weights(c2_w)
    wf = _build_fc1_weights(fc1_w)
    w2f = fc2_w.astype(jnp.bfloat16)
    w3f = fc3_w.astype(jnp.bfloat16)

    full = lambda s: pl.BlockSpec(s, lambda i: (0,) * len(s))
    out = pl.pallas_call(
        _fused_kernel,
        out_shape=jax.ShapeDtypeStruct((B, 128), jnp.float32),
        grid=(B // bt,),
        in_specs=[
            pl.BlockSpec((bt, 3, 32, 32), lambda i: (i, 0, 0, 0)),
            full((5, 96, 256)), full((1, 128)),
            full((6, 256, 256)), full((1, 128)),
            full((3, 256, 128)), full((1, 128)),
            full((128, 128)), full((1, 128)),
            full((128, 128)), full((1, 128)),
        ],
        out_specs=pl.BlockSpec((bt, 128), lambda i: (i, 0)),
        scratch_shapes=[
            pltpu.VMEM((bt * 32 + 8, 96), jnp.bfloat16),
            pltpu.VMEM((bt, 32, 128), jnp.bfloat16),
            pltpu.VMEM((bt, 8, 256), jnp.bfloat16),
        ],
        compiler_params=pltpu.CompilerParams(
            dimension_semantics=("parallel",),
            vmem_limit_bytes=48 * 1024 * 1024),
    )(x, w1, c1_b, w2, c2_b, wf, fc1_b, w2f, fc2_b, w3f, fc3_b)
    return out[:, :10]


# speedup vs baseline: 2.0053x; 2.0053x over previous
import numpy as np

import jax
import jax.numpy as jnp
from jax.experimental import pallas as pl
from jax.experimental.pallas import tpu as pltpu


_A = np.zeros((2, 2, 2, 4, 5), np.float32)
for _pe in range(2):
    for _p in range(2):
        for _q in range(2):
            for _rq in range(4):
                _di = 4 * _q + _rq - 2 * _pe - _p
                if 0 <= _di < 5:
                    _A[_pe, _p, _q, _rq, _di] = 1.0
_B = np.zeros((32, 2, 14, 5), np.float32)
for _br in range(2):
    for _col2 in range(14):
        for _dj in range(5):
            _ci = 2 * _col2 + _br + _dj
            if _ci < 32:
                _B[_ci, _br, _col2, _dj] = 1.0


def _build_conv1_weights(c1_w):
    w4 = c1_w[:, :15, :6].reshape(5, 5, 3, 6)
    e = jnp.einsum("PpQRD,CbKJ,DJco->PpcQRCbKo", _A, _B, w4)
    e = jnp.pad(e, ((0, 0),) * 8 + ((0, 2),))
    e = e.reshape(2, 2, 3, 256, 2, 112)
    e = jnp.pad(e, ((0, 0),) * 5 + ((0, 16),))
    return e.reshape(12, 256, 256).astype(jnp.bfloat16)


def _build_conv2_weights(c2_w):
    z = jnp.zeros((128, 256), c2_w.dtype)
    w = jnp.stack([
        jnp.concatenate([c2_w[0], c2_w[1]], axis=0),
        jnp.concatenate([c2_w[2], c2_w[3]], axis=0),
        jnp.concatenate([c2_w[4], z], axis=0),
        jnp.concatenate([z, c2_w[0]], axis=0),
        jnp.concatenate([c2_w[1], c2_w[2]], axis=0),
        jnp.concatenate([c2_w[3], c2_w[4]], axis=0),
    ])
    return w.astype(jnp.bfloat16)


def _build_fc1_weights(fc1_w):
    def ch(r):
        return jnp.pad(fc1_w[80 * r:80 * r + 80], ((0, 48), (0, 0)))
    z = jnp.zeros((128, 128), fc1_w.dtype)
    w = jnp.stack([
        jnp.concatenate([ch(0), ch(1)], axis=0),
        jnp.concatenate([ch(2), ch(3)], axis=0),
        jnp.concatenate([ch(4), z], axis=0),
    ])
    return w.astype(jnp.bfloat16)


def _fused_kernel(x_ref, w1_ref, b1_ref, w2_ref, b2_ref, wf_ref, bf1_ref,
                  w2f_ref, bf2_ref, w3f_ref, bf3_ref, o_ref, xsc, h1sc):
    bt = x_ref.shape[0]
    R = bt * 8
    f32 = jnp.float32
    bf16 = jnp.bfloat16

    for c in range(3):
        xc = x_ref[:, c].reshape(R, 128)
        xsc[pl.ds(1, R), pl.ds(256 * c, 128)] = xc
        xsc[pl.ds(0, R), pl.ds(256 * c + 128, 128)] = xc

    lhs = [xsc[pl.ds(1, R - 1), pl.ds(256 * c, 256)] for c in range(3)]
    halves = []
    for pe in range(2):
        ms = None
        for p in range(2):
            acc = None
            for c in range(3):
                d = jnp.dot(lhs[c], w1_ref[(pe * 2 + p) * 3 + c],
                            preferred_element_type=f32)
                acc = d if acc is None else acc + d
            ms = acc if ms is None else jnp.maximum(ms, acc)
        m = jnp.maximum(ms[:, :128], ms[:, 128:])
        halves.append(jnp.maximum(m + b1_ref[...], 0.0).astype(bf16))
    h1sc[pl.ds(R - 1, 3), :] = jnp.zeros((3, 256), bf16)
    h1sc[pl.ds(0, R - 1), pl.ds(0, 128)] = halves[0]
    h1sc[pl.ds(0, R - 1), pl.ds(128, 128)] = halves[1]

    accs = [None, None]
    for o in range(3):
        sl = h1sc[pl.ds(o, R - 2), :]
        for p in range(2):
            d = jnp.dot(sl, w2_ref[p * 3 + o], preferred_element_type=f32)
            accs[p] = d if accs[p] is None else accs[p] + d
    m2 = jnp.maximum(accs[0], accs[1])
    m2 = jnp.maximum(m2[:, :128], m2[:, 128:])
    h2 = jnp.maximum(m2 + b2_ref[...], 0.0).astype(bf16)
    h2 = jnp.concatenate([h2, jnp.zeros((2, 128), bf16)], axis=0)
    h2 = h2.reshape(bt, 8, 128)

    f = None
    for r in range(3):
        l = jnp.concatenate([h2[:, 2 * r, :], h2[:, 2 * r + 1, :]], axis=1)
        d = jnp.dot(l, wf_ref[r], preferred_element_type=f32)
        f = d if f is None else f + d
    h = jnp.maximum(f + bf1_ref[...], 0.0).astype(bf16)
    h = jnp.dot(h, w2f_ref[...], preferred_element_type=f32)
    h = jnp.maximum(h + bf2_ref[...], 0.0).astype(bf16)
    out = jnp.dot(h, w3f_ref[...], preferred_element_type=f32) + bf3_ref[...]
    o_ref[...] = out


def kernel(c1_w, c1_b, c2_w, c2_b, fc1_w, fc1_b, fc2_w, fc2_b, fc3_w, fc3_b,
           x):
    B = x.shape[0]
    bt = next(b for b in (128, 64, 32, 16, 8, 4, 2, 1) if B % b == 0)
    x4 = x.reshape(B, 3, 8, 128).astype(jnp.bfloat16)

    w1 = _build_conv1_weights(c1_w)
    w2 = _build_conv2_weights(c2_w)
    wf = _build_fc1_weights(fc1_w)
    w2f = fc2_w.astype(jnp.bfloat16)
    w3f = fc3_w.astype(jnp.bfloat16)

    full = lambda s: pl.BlockSpec(s, lambda i: (0,) * len(s))
    out = pl.pallas_call(
        _fused_kernel,
        out_shape=jax.ShapeDtypeStruct((B, 128), jnp.float32),
        grid=(B // bt,),
        in_specs=[
            pl.BlockSpec((bt, 3, 8, 128), lambda i: (i, 0, 0, 0)),
            full((12, 256, 256)), full((1, 128)),
            full((6, 256, 256)), full((1, 128)),
            full((3, 256, 128)), full((1, 128)),
            full((128, 128)), full((1, 128)),
            full((128, 128)), full((1, 128)),
        ],
        out_specs=pl.BlockSpec((bt, 128), lambda i: (i, 0)),
        scratch_shapes=[
            pltpu.VMEM((bt * 8 + 1, 768), jnp.bfloat16),
            pltpu.VMEM((bt * 8 + 2, 256), jnp.bfloat16),
        ],
        compiler_params=pltpu.CompilerParams(
            dimension_semantics=("parallel",),
            vmem_limit_bytes=48 * 1024 * 1024),
    )(x4, w1, c1_b, w2, c2_b, wf, fc1_b, w2f, fc2_b, w3f, fc3_b)
    return out[:, :10]


# speedup vs baseline: 2.1624x; 1.0783x over previous
import numpy as np

import jax
import jax.numpy as jnp
from jax.experimental import pallas as pl
from jax.experimental.pallas import tpu as pltpu


_A = np.zeros((2, 2, 2, 4, 5), np.float32)
for _pe in range(2):
    for _p in range(2):
        for _q in range(2):
            for _rq in range(4):
                _di = 4 * _q + _rq - 2 * _pe - _p
                if 0 <= _di < 5:
                    _A[_pe, _p, _q, _rq, _di] = 1.0
_B = np.zeros((32, 2, 14, 5), np.float32)
for _br in range(2):
    for _col2 in range(14):
        for _dj in range(5):
            _ci = 2 * _col2 + _br + _dj
            if _ci < 32:
                _B[_ci, _br, _col2, _dj] = 1.0


def _build_conv1_weights(c1_w):
    w4 = c1_w[:, :15, :6].reshape(5, 5, 3, 6)
    e = jnp.einsum("PpQRD,CbKJ,DJco->PpcQRCbKo", _A, _B, w4)
    e = jnp.pad(e, ((0, 0),) * 8 + ((0, 2),))
    e = e.reshape(2, 2, 3, 256, 2, 112)
    e = jnp.pad(e, ((0, 0),) * 5 + ((0, 16),))
    return e.reshape(12, 256, 256).astype(jnp.bfloat16)


def _build_conv2_weights(c2_w):
    z = jnp.zeros((128, 256), c2_w.dtype)
    w = jnp.stack([
        jnp.concatenate([c2_w[0], c2_w[1]], axis=0),
        jnp.concatenate([c2_w[2], c2_w[3]], axis=0),
        jnp.concatenate([c2_w[4], z], axis=0),
        jnp.concatenate([z, c2_w[0]], axis=0),
        jnp.concatenate([c2_w[1], c2_w[2]], axis=0),
        jnp.concatenate([c2_w[3], c2_w[4]], axis=0),
    ])
    return w.astype(jnp.bfloat16)


def _build_fc1_weights(fc1_w):
    def ch(r):
        return jnp.pad(fc1_w[80 * r:80 * r + 80], ((0, 48), (0, 0)))
    z = jnp.zeros((128, 128), fc1_w.dtype)
    w = jnp.stack([
        jnp.concatenate([ch(0), ch(1)], axis=0),
        jnp.concatenate([ch(2), ch(3)], axis=0),
        jnp.concatenate([ch(4), z], axis=0),
    ])
    return w.astype(jnp.bfloat16)


def _fused_kernel(x_ref, w1_ref, b1_ref, w2_ref, b2_ref, wf_ref, bf1_ref,
                  w2f_ref, bf2_ref, w3f_ref, bf3_ref, o_ref):
    bt = x_ref.shape[0]
    R = bt * 8
    f32 = jnp.float32
    bf16 = jnp.bfloat16

    lhs = []
    for c in range(3):
        xc = x_ref[:, c].reshape(R, 128).astype(bf16)
        lhs.append(jnp.concatenate([xc[:R - 1], xc[1:]], axis=1))
    halves = []
    for pe in range(2):
        ms = None
        for p in range(2):
            acc = None
            for c in range(3):
                d = jnp.dot(lhs[c], w1_ref[(pe * 2 + p) * 3 + c],
                            preferred_element_type=f32)
                acc = d if acc is None else acc + d
            ms = acc if ms is None else jnp.maximum(ms, acc)
        m = jnp.maximum(ms[:, :128], ms[:, 128:])
        halves.append(jnp.maximum(m + b1_ref[...], 0.0))
    h1 = jnp.concatenate(halves, axis=1).astype(bf16)
    h1 = jnp.concatenate([h1, jnp.zeros((1, 256), bf16)], axis=0)

    accs = [None, None]
    for o in range(3):
        sl = h1[o:R - 2 + o]
        for p in range(2):
            d = jnp.dot(sl, w2_ref[p * 3 + o], preferred_element_type=f32)
            accs[p] = d if accs[p] is None else accs[p] + d
    m2 = jnp.maximum(accs[0], accs[1])
    m2 = jnp.maximum(m2[:, :128], m2[:, 128:])
    h2 = jnp.maximum(m2 + b2_ref[...], 0.0).astype(bf16)
    h2 = jnp.concatenate([h2, jnp.zeros((2, 128), bf16)], axis=0)
    h2 = h2.reshape(bt, 8, 128)

    f = None
    for r in range(3):
        l = jnp.concatenate([h2[:, 2 * r, :], h2[:, 2 * r + 1, :]], axis=1)
        d = jnp.dot(l, wf_ref[r], preferred_element_type=f32)
        f = d if f is None else f + d
    h = jnp.maximum(f + bf1_ref[...], 0.0).astype(bf16)
    h = jnp.dot(h, w2f_ref[...], preferred_element_type=f32)
    h = jnp.maximum(h + bf2_ref[...], 0.0).astype(bf16)
    out = jnp.dot(h, w3f_ref[...], preferred_element_type=f32) + bf3_ref[...]
    o_ref[...] = out


def kernel(c1_w, c1_b, c2_w, c2_b, fc1_w, fc1_b, fc2_w, fc2_b, fc3_w, fc3_b,
           x):
    B = x.shape[0]
    bt = next(b for b in (256, 128, 64, 32, 16, 8, 4, 2, 1) if B % b == 0)
    x4 = x.reshape(B, 3, 8, 128)

    w1 = _build_conv1_weights(c1_w)
    w2 = _build_conv2_weights(c2_w)
    wf = _build_fc1_weights(fc1_w)
    w2f = fc2_w.astype(jnp.bfloat16)
    w3f = fc3_w.astype(jnp.bfloat16)

    full = lambda s: pl.BlockSpec(s, lambda i: (0,) * len(s))
    out = pl.pallas_call(
        _fused_kernel,
        out_shape=jax.ShapeDtypeStruct((B, 128), jnp.float32),
        grid=(B // bt,),
        in_specs=[
            pl.BlockSpec((bt, 3, 8, 128), lambda i: (i, 0, 0, 0)),
            full((12, 256, 256)), full((1, 128)),
            full((6, 256, 256)), full((1, 128)),
            full((3, 256, 128)), full((1, 128)),
            full((128, 128)), full((1, 128)),
            full((128, 128)), full((1, 128)),
        ],
        out_specs=pl.BlockSpec((bt, 128), lambda i: (i, 0)),
        compiler_params=pltpu.CompilerParams(
            dimension_semantics=("parallel",),
            vmem_limit_bytes=48 * 1024 * 1024),
    )(x4, w1, c1_b, w2, c2_b, wf, fc1_b, w2f, fc2_b, w3f, fc3_b)
    return out[:, :10]


# speedup vs baseline: 2.2027x; 1.0187x over previous
import numpy as np

import jax
import jax.numpy as jnp
from jax.experimental import pallas as pl
from jax.experimental.pallas import tpu as pltpu


_A = np.zeros((2, 2, 2, 4, 5), np.float32)
for _pe in range(2):
    for _p in range(2):
        for _q in range(2):
            for _rq in range(4):
                _di = 4 * _q + _rq - 2 * _pe - _p
                if 0 <= _di < 5:
                    _A[_pe, _p, _q, _rq, _di] = 1.0
_B = np.zeros((32, 2, 14, 5), np.float32)
for _br in range(2):
    for _col2 in range(14):
        for _dj in range(5):
            _ci = 2 * _col2 + _br + _dj
            if _ci < 32:
                _B[_ci, _br, _col2, _dj] = 1.0


def _build_conv1_weights(c1_w):
    w4 = c1_w[:, :15, :6].reshape(5, 5, 3, 6)
    e = jnp.einsum("PpQRD,CbKJ,DJco->PpcQRCbKo", _A, _B, w4)
    e = jnp.pad(e, ((0, 0),) * 8 + ((0, 2),))
    e = e.reshape(2, 2, 3, 256, 2, 112)
    e = jnp.pad(e, ((0, 0),) * 5 + ((0, 16),))
    return e.reshape(12, 256, 256).astype(jnp.bfloat16)


def _build_conv2_weights(c2_w):
    z = jnp.zeros((128, 256), c2_w.dtype)
    w = jnp.stack([
        jnp.concatenate([c2_w[0], c2_w[1]], axis=0),
        jnp.concatenate([c2_w[2], c2_w[3]], axis=0),
        jnp.concatenate([c2_w[4], z], axis=0),
        jnp.concatenate([z, c2_w[0]], axis=0),
        jnp.concatenate([c2_w[1], c2_w[2]], axis=0),
        jnp.concatenate([c2_w[3], c2_w[4]], axis=0),
    ])
    return w.astype(jnp.bfloat16)


def _build_fc1_weights(fc1_w):
    def ch(r):
        return jnp.pad(fc1_w[80 * r:80 * r + 80], ((0, 48), (0, 0)))
    z = jnp.zeros((128, 128), fc1_w.dtype)
    w = jnp.stack([
        jnp.concatenate([ch(0), ch(1)], axis=0),
        jnp.concatenate([ch(2), ch(3)], axis=0),
        jnp.concatenate([ch(4), z], axis=0),
    ])
    return w.astype(jnp.bfloat16)


def _fused_kernel(x_ref, w1_ref, b1_ref, w2_ref, b2_ref, wf_ref, bf1_ref,
                  w2f_ref, bf2_ref, w3f_ref, bf3_ref, o_ref):
    bt = x_ref.shape[0]
    R = bt * 8
    f32 = jnp.float32
    bf16 = jnp.bfloat16

    lhs = []
    for c in range(3):
        xc = x_ref[:, c].reshape(R, 128).astype(bf16)
        lhs.append(jnp.concatenate([xc[:R - 1], xc[1:]], axis=1))
    halves = []
    for pe in range(2):
        ms = None
        for p in range(2):
            acc = None
            for c in range(3):
                d = jnp.dot(lhs[c], w1_ref[(pe * 2 + p) * 3 + c],
                            preferred_element_type=f32)
                acc = d if acc is None else acc + d
            ms = acc if ms is None else jnp.maximum(ms, acc)
        m = jnp.maximum(ms[:, :128], ms[:, 128:])
        halves.append(jnp.maximum(m + b1_ref[...], 0.0))
    h1 = jnp.concatenate(halves, axis=1).astype(bf16)
    h1 = jnp.concatenate([h1, jnp.zeros((1, 256), bf16)], axis=0)

    accs = [None, None]
    for o in range(3):
        sl = h1[o:R - 2 + o]
        for p in range(2):
            d = jnp.dot(sl, w2_ref[p * 3 + o], preferred_element_type=f32)
            accs[p] = d if accs[p] is None else accs[p] + d
    m2 = jnp.maximum(accs[0], accs[1])
    m2 = jnp.maximum(m2[:, :128], m2[:, 128:])
    h2 = jnp.maximum(m2 + b2_ref[...], 0.0).astype(bf16)
    h2 = jnp.concatenate([h2, jnp.zeros((2, 128), bf16)], axis=0)
    h2 = h2.reshape(bt, 8, 128)

    f = None
    for r in range(3):
        l = jnp.concatenate([h2[:, 2 * r, :], h2[:, 2 * r + 1, :]], axis=1)
        d = jnp.dot(l, wf_ref[r], preferred_element_type=f32)
        f = d if f is None else f + d
    h = jnp.maximum(f + bf1_ref[...], 0.0).astype(bf16)
    h = jnp.dot(h, w2f_ref[...], preferred_element_type=f32)
    h = jnp.maximum(h + bf2_ref[...], 0.0).astype(bf16)
    out = jnp.dot(h, w3f_ref[...], preferred_element_type=f32) + bf3_ref[...]
    o_ref[...] = out


def kernel(c1_w, c1_b, c2_w, c2_b, fc1_w, fc1_b, fc2_w, fc2_b, fc3_w, fc3_b,
           x):
    B = x.shape[0]
    bt = next(b for b in (512, 256, 128, 64, 32, 16, 8, 4, 2, 1) if B % b == 0)
    x4 = x.reshape(B, 3, 8, 128)

    w1 = _build_conv1_weights(c1_w)
    w2 = _build_conv2_weights(c2_w)
    wf = _build_fc1_weights(fc1_w)
    w2f = fc2_w.astype(jnp.bfloat16)
    w3f = fc3_w.astype(jnp.bfloat16)

    full = lambda s: pl.BlockSpec(s, lambda i: (0,) * len(s))
    out = pl.pallas_call(
        _fused_kernel,
        out_shape=jax.ShapeDtypeStruct((B, 128), jnp.float32),
        grid=(B // bt,),
        in_specs=[
            pl.BlockSpec((bt, 3, 8, 128), lambda i: (i, 0, 0, 0)),
            full((12, 256, 256)), full((1, 128)),
            full((6, 256, 256)), full((1, 128)),
            full((3, 256, 128)), full((1, 128)),
            full((128, 128)), full((1, 128)),
            full((128, 128)), full((1, 128)),
        ],
        out_specs=pl.BlockSpec((bt, 128), lambda i: (i, 0)),
        compiler_params=pltpu.CompilerParams(
            dimension_semantics=("parallel",),
            vmem_limit_bytes=48 * 1024 * 1024),
    )(x4, w1, c1_b, w2, c2_b, wf, fc1_b, w2f, fc2_b, w3f, fc3_b)
    return out[:, :10]
